# baseline (speedup 1.0000x reference)
import functools

import jax
import jax.numpy as jnp
from jax import lax
from jax.experimental import pallas as pl
from jax.experimental.pallas import tpu as pltpu
from jax.experimental.pallas import tpu_sc as plsc

_NF = 256
_NC = 1024
_B = 8
_HW = 24 * 24
_N = _B * _HW
_BLK = 512
_NBLK = _N // _BLK

_NW = 32
_FPW = _NF // _NW
_L = 16


def _tc_body(x_ref, c_ref, idx_ref, loss_ref):
    i = pl.program_id(0)
    x = x_ref[...]
    c = c_ref[...]
    mm = jnp.dot(x, c, preferred_element_type=jnp.float32)
    c_sq = jnp.sum(c * c, axis=0, keepdims=True)
    x_sq = jnp.sum(x * x, axis=1, keepdims=True)
    dist = (c_sq + x_sq) - 2.0 * mm
    m = jnp.min(dist, axis=1, keepdims=True)
    ids = lax.broadcasted_iota(jnp.int32, dist.shape, 1)
    idx = jnp.min(jnp.where(dist == m, ids, _NC), axis=1)
    idx_ref[0, 0, :] = idx

    @pl.when(i == 0)
    def _():
        loss_ref[...] = jnp.zeros((1, 1), jnp.float32)

    loss_ref[...] += jnp.sum(m, axis=(0, 1), keepdims=True)


_tc_call = pl.pallas_call(
    _tc_body,
    grid=(_NBLK,),
    in_specs=[
        pl.BlockSpec((_BLK, _NF), lambda i: (i, 0)),
        pl.BlockSpec((_NF, _NC), lambda i: (0, 0)),
    ],
    out_specs=[
        pl.BlockSpec((1, 1, _BLK), lambda i: (i, 0, 0)),
        pl.BlockSpec((1, 1), lambda i: (0, 0)),
    ],
    out_shape=[
        jax.ShapeDtypeStruct((_NBLK, 1, _BLK), jnp.int32),
        jax.ShapeDtypeStruct((1, 1), jnp.float32),
    ],
)


@functools.cache
def _sc_gather_call():
    @functools.partial(
        pl.kernel,
        mesh=plsc.VectorSubcoreMesh(core_axis_name="c", subcore_axis_name="s"),
        out_type=jax.ShapeDtypeStruct((_B, _NF, _HW), jnp.float32),
        scratch_types=[
            pltpu.VMEM((_FPW, _NC), jnp.float32),
            pltpu.VMEM((_N,), jnp.int32),
            pltpu.VMEM((_B, _FPW, _HW), jnp.float32),
        ],
        compiler_params=pltpu.CompilerParams(
            needs_layout_passes=False, use_tc_tiling_on_sc=False
        ),
    )
    def _sc_gather(c_hbm, idx_hbm, out_hbm, tab_v, idx_v, vals_v):
        wid = lax.axis_index("s") * 2 + lax.axis_index("c")
        f0 = wid * _FPW
        pltpu.sync_copy(c_hbm.at[pl.ds(f0, _FPW)], tab_v)
        pltpu.sync_copy(idx_hbm, idx_v)

        iota = lax.iota(jnp.int32, _L)
        for b in range(_B):
            def body(k, carry, b=b):
                s = k * _L + iota
                perm = (s % 24) * 24 + s // 24 + b * _HW
                iv = plsc.load_gather(idx_v, [perm])
                for f in range(_FPW):
                    fv = jnp.full((_L,), f, jnp.int32)
                    vals_v[b, f, pl.ds(k * _L, _L)] = plsc.load_gather(
                        tab_v, [fv, iv]
                    )
                return carry

            lax.fori_loop(0, _HW // _L, body, 0, unroll=2)
        for b in range(_B):
            pltpu.sync_copy(
                vals_v.at[b],
                out_hbm.at[b, pl.ds(f0, _FPW), :],
            )

    return _sc_gather


def kernel(x, centroids):
    x_flat = jnp.swapaxes(x, 1, -1).reshape(_N, _NF)
    idx, loss_sum = _tc_call(x_flat, centroids)
    x_q = _sc_gather_call()(centroids, idx.reshape(_N))
    x_q = x_q.reshape(x.shape)
    loss = loss_sum[0, 0] / jnp.float32(x.size)
    return x_q, loss


# speedup vs baseline: 1.1652x; 1.1652x over previous
import functools

import jax
import jax.numpy as jnp
from jax import lax
from jax.experimental import pallas as pl
from jax.experimental.pallas import tpu as pltpu
from jax.experimental.pallas import tpu_sc as plsc

_NF = 256
_NC = 1024
_B = 8
_HW = 24 * 24
_N = _B * _HW
_BLK = 512
_NBLK = _N // _BLK

_NW = 32
_FPW = _NF // _NW
_L = 16


def _tc_body(x_ref, c_ref, idx_ref, loss_ref):
    i = pl.program_id(0)
    x = x_ref[...]
    c = c_ref[...]
    mm = jnp.dot(x, c, preferred_element_type=jnp.float32)
    c_sq = jnp.sum(c * c, axis=0, keepdims=True)
    x_sq = jnp.sum(x * x, axis=1, keepdims=True)
    dist = (c_sq + x_sq) - 2.0 * mm
    m = jnp.min(dist, axis=1, keepdims=True)
    ids = lax.broadcasted_iota(jnp.int32, dist.shape, 1)
    idx = jnp.min(jnp.where(dist == m, ids, _NC), axis=1)
    idx_ref[0, 0, :] = idx

    @pl.when(i == 0)
    def _():
        loss_ref[...] = jnp.zeros((1, 1), jnp.float32)

    loss_ref[...] += jnp.sum(m, axis=(0, 1), keepdims=True)

    @pl.when(i == _NBLK - 1)
    def _():
        loss_ref[...] = loss_ref[...] * (1.0 / float(_N * _NF))


_tc_call = pl.pallas_call(
    _tc_body,
    grid=(_NBLK,),
    in_specs=[
        pl.BlockSpec((_BLK, _NF), lambda i: (i, 0)),
        pl.BlockSpec((_NF, _NC), lambda i: (0, 0)),
    ],
    out_specs=[
        pl.BlockSpec((1, 1, _BLK), lambda i: (i, 0, 0)),
        pl.BlockSpec((1, 1), lambda i: (0, 0)),
    ],
    out_shape=[
        jax.ShapeDtypeStruct((_NBLK, 1, _BLK), jnp.int32),
        jax.ShapeDtypeStruct((1, 1), jnp.float32),
    ],
)


@functools.cache
def _sc_gather_call():
    @functools.partial(
        pl.kernel,
        mesh=plsc.VectorSubcoreMesh(core_axis_name="c", subcore_axis_name="s"),
        out_type=jax.ShapeDtypeStruct((_B, _NF, _HW), jnp.float32),
        scratch_types=[
            pltpu.VMEM((_FPW, _NC), jnp.float32),
            pltpu.VMEM((_N,), jnp.int32),
            pltpu.VMEM((_B, _FPW, _HW), jnp.float32),
        ],
        compiler_params=pltpu.CompilerParams(
            needs_layout_passes=False, use_tc_tiling_on_sc=False
        ),
    )
    def _sc_gather(c_hbm, idx_hbm, out_hbm, tab_v, idx_v, vals_v):
        wid = lax.axis_index("s") * 2 + lax.axis_index("c")
        f0 = wid * _FPW
        pltpu.sync_copy(c_hbm.at[pl.ds(f0, _FPW)], tab_v)
        pltpu.sync_copy(idx_hbm, idx_v)

        iota = lax.iota(jnp.int32, _L)
        fvs = [jnp.full((_L,), f, jnp.int32) for f in range(_FPW)]
        for b in range(_B):
            @plsc.parallel_loop(0, _HW, _L, unroll=2)
            def body(i, b=b):
                s = i + iota
                perm = (s % 24) * 24 + s // 24 + b * _HW
                iv = plsc.load_gather(idx_v, [perm])
                for f in range(_FPW):
                    vals_v[b, f, pl.ds(i, _L)] = plsc.load_gather(
                        tab_v, [fvs[f], iv]
                    )
        for b in range(_B):
            pltpu.sync_copy(
                vals_v.at[b],
                out_hbm.at[b, pl.ds(f0, _FPW), :],
            )

    return _sc_gather


def kernel(x, centroids):
    x_flat = jnp.swapaxes(x, 1, -1).reshape(_N, _NF)
    idx, loss_sum = _tc_call(x_flat, centroids)
    x_q = _sc_gather_call()(centroids, idx.reshape(_N))
    x_q = x_q.reshape(x.shape)
    return x_q, loss_sum[0, 0]


# speedup vs baseline: 1.2810x; 1.0994x over previous
import functools

import jax
import jax.numpy as jnp
from jax import lax
from jax.experimental import pallas as pl
from jax.experimental.pallas import tpu as pltpu
from jax.experimental.pallas import tpu_sc as plsc

_NF = 256
_NC = 1024
_B = 8
_HW = 24 * 24
_N = _B * _HW

_NW = 32
_FPW = _NF // _NW
_L = 16


def _tc_body(x_ref, c_ref, idx_ref, loss_ref):
    b = pl.program_id(0)

    @pl.when(b == 0)
    def _():
        loss_ref[...] = jnp.zeros((1, 1), jnp.float32)

    c = c_ref[...]
    x = x_ref[0]
    mm = lax.dot_general(
        c, x,
        (((0,), (0,)), ((), ())),
        preferred_element_type=jnp.float32,
    )
    c_sq = jnp.sum(c * c, axis=0)[:, None]
    x_sq = jnp.sum(x * x, axis=0, keepdims=True)
    dist = (c_sq + x_sq) - 2.0 * mm
    m = jnp.min(dist, axis=0, keepdims=True)
    ids = lax.broadcasted_iota(jnp.int32, dist.shape, 0)
    idx = jnp.min(jnp.where(dist == m, ids, _NC), axis=0)
    idx_ref[0, 0, :] = idx

    loss_ref[...] += jnp.sum(m, axis=(0, 1), keepdims=True)

    @pl.when(b == _B - 1)
    def _():
        loss_ref[...] = loss_ref[...] * (1.0 / float(_N * _NF))


_tc_call = pl.pallas_call(
    _tc_body,
    grid=(_B,),
    in_specs=[
        pl.BlockSpec((1, _NF, _HW), lambda b: (b, 0, 0)),
        pl.BlockSpec((_NF, _NC), lambda b: (0, 0)),
    ],
    out_specs=[
        pl.BlockSpec((1, 1, _HW), lambda b: (b, 0, 0)),
        pl.BlockSpec((1, 1), lambda b: (0, 0)),
    ],
    out_shape=[
        jax.ShapeDtypeStruct((_B, 1, _HW), jnp.int32),
        jax.ShapeDtypeStruct((1, 1), jnp.float32),
    ],
)


@functools.cache
def _sc_gather_call():
    @functools.partial(
        pl.kernel,
        mesh=plsc.VectorSubcoreMesh(core_axis_name="c", subcore_axis_name="s"),
        out_type=jax.ShapeDtypeStruct((_B, _NF, _HW), jnp.float32),
        scratch_types=[
            pltpu.VMEM((_FPW, _NC), jnp.float32),
            pltpu.VMEM((_N,), jnp.int32),
            pltpu.VMEM((_B, _FPW, _HW), jnp.float32),
        ],
        compiler_params=pltpu.CompilerParams(
            needs_layout_passes=False, use_tc_tiling_on_sc=False
        ),
    )
    def _sc_gather(c_hbm, idx_hbm, out_hbm, tab_v, idx_v, vals_v):
        wid = lax.axis_index("s") * 2 + lax.axis_index("c")
        f0 = wid * _FPW
        pltpu.sync_copy(c_hbm.at[pl.ds(f0, _FPW)], tab_v)
        pltpu.sync_copy(idx_hbm, idx_v)

        iota = lax.iota(jnp.int32, _L)
        fvs = [jnp.full((_L,), f, jnp.int32) for f in range(_FPW)]
        for b in range(_B):
            @plsc.parallel_loop(0, _HW, _L, unroll=2)
            def body(i, b=b):
                iv = plsc.load_gather(idx_v, [i + iota + b * _HW])
                for f in range(_FPW):
                    vals_v[b, f, pl.ds(i, _L)] = plsc.load_gather(
                        tab_v, [fvs[f], iv]
                    )
        for b in range(_B):
            pltpu.sync_copy(
                vals_v.at[b],
                out_hbm.at[b, pl.ds(f0, _FPW), :],
            )

    return _sc_gather


def kernel(x, centroids):
    x_r = x.reshape(_B, _NF, _HW)
    idx, loss_sum = _tc_call(x_r, centroids)
    x_q = _sc_gather_call()(centroids, idx.reshape(_N))
    x_q = x_q.reshape(x.shape)
    return x_q, loss_sum[0, 0]
